# baseline (speedup 1.0000x reference)
import jax
import jax.numpy as jnp
from jax.experimental import pallas as pl
from jax.experimental.pallas import tpu as pltpu

_K = 512
_D = 64
_NUM_CLASSES = 60
_N = 1024


def _vq_body(c_ref, z_hbm, cbt_hbm, out_ref, z_buf, cb_buf, z_sem, cb_sem):
    b = pl.program_id(0)
    nb = pl.num_programs(0)
    slot = jax.lax.rem(b, 4)
    nxt = jax.lax.rem(b + 3, 4)

    def _start(i, s):
        pltpu.make_async_copy(z_hbm.at[i], z_buf.at[s], z_sem.at[s]).start()
        pltpu.make_async_copy(
            cbt_hbm.at[:, pl.ds(c_ref[i] * _K, _K)], cb_buf.at[s], cb_sem.at[s]
        ).start()

    @pl.when(b == 0)
    def _():
        _start(0, 0)
        _start(1, 1)
        _start(2, 2)

    @pl.when(b + 3 < nb)
    def _():
        _start(b + 3, nxt)

    pltpu.make_async_copy(z_hbm.at[b], z_buf.at[slot], z_sem.at[slot]).wait()
    pltpu.make_async_copy(
        cbt_hbm.at[:, pl.ds(c_ref[b] * _K, _K)], cb_buf.at[slot], cb_sem.at[slot]
    ).wait()

    z = z_buf[slot]
    subt = cb_buf[slot]
    e_sq = jnp.sum(subt * subt, axis=0, keepdims=True)
    subt2 = -2.0 * subt
    cross2 = jax.lax.dot_general(
        z, subt2, (((1,), (0,)), ((), ())),
        preferred_element_type=jnp.float32)
    dist = cross2 + e_sq
    minv = jnp.min(dist, axis=1, keepdims=True)
    iota = jax.lax.broadcasted_iota(jnp.int32, (_N, _K), 1)
    idx = jnp.min(jnp.where(dist == minv, iota, _K), axis=1, keepdims=True)
    onehot = (iota == idx).astype(jnp.float32)
    quant = jax.lax.dot_general(
        onehot, subt, (((1,), (1,)), ((), ())),
        preferred_element_type=jnp.float32)
    out_ref[0] = quant


def kernel(z_e_x, c, emb_weight):
    B = z_e_x.shape[0]
    zf = jnp.transpose(z_e_x, (0, 2, 3, 1)).reshape(B, _N, _D)
    cbt = jnp.transpose(emb_weight, (1, 0))
    zf = pltpu.with_memory_space_constraint(zf, pltpu.MemorySpace.HBM)
    cbt = pltpu.with_memory_space_constraint(cbt, pltpu.MemorySpace.HBM)
    grid_spec = pltpu.PrefetchScalarGridSpec(
        num_scalar_prefetch=1,
        grid=(B,),
        in_specs=[
            pl.BlockSpec(memory_space=pltpu.MemorySpace.HBM),
            pl.BlockSpec(memory_space=pltpu.MemorySpace.HBM),
        ],
        out_specs=pl.BlockSpec((1, _N, _D), lambda b, c_ref: (b, 0, 0)),
        scratch_shapes=[
            pltpu.VMEM((4, _N, _D), jnp.float32),
            pltpu.VMEM((4, _D, _K), jnp.float32),
            pltpu.SemaphoreType.DMA((4,)),
            pltpu.SemaphoreType.DMA((4,)),
        ],
    )
    out = pl.pallas_call(
        _vq_body,
        grid_spec=grid_spec,
        out_shape=jax.ShapeDtypeStruct((B, _N, _D), jnp.float32),
    )(c, zf, cbt)
    return out.reshape(B, 32, 32, _D)
